# baseline (speedup 1.0000x reference)
import functools

import numpy as np
import jax
import jax.numpy as jnp
from jax import lax
from jax.experimental import pallas as pl
from jax.experimental.pallas import tpu as pltpu
from jax.experimental.pallas import tpu_sc as plsc

N = 10000
E = 320000
H = 128
DB = 16
G = 256
NL = 3

NS = 16
CH = 64
CHUNKS = E // CH
TRIPS = (CHUNKS + NS - 1) // NS
RPT = 624
RPT_LAST = N - (NS - 1) * RPT

_F32 = jnp.float32


def _sc_edge_body(xw, ew, src, dst, agg_out, src_v, dst_v, ew_buf, g_buf,
                  m_buf, agg_sp, sem_i0, sem_i1, sem_e0, sem_e1, sem_g,
                  sem_s0, sem_s1):
    c = lax.axis_index("c")
    s = lax.axis_index("s")
    sem_i = (sem_i0, sem_i1)
    sem_e = (sem_e0, sem_e1)
    sem_s = (sem_s0, sem_s1)

    def _zrow(i, carry):
        for v in range(H // 16):
            m_buf[0, i, pl.ds(v * 16, 16)] = jnp.zeros((16,), _F32)
        return carry
    lax.fori_loop(0, CH, _zrow, 0)
    row0 = s * RPT

    @pl.when(s < NS - 1)
    def _zero_mid():
        off = 0
        while off < RPT:
            sz = min(CH, RPT - off)
            pltpu.sync_copy(m_buf.at[0].at[pl.ds(0, sz)],
                            agg_sp.at[pl.ds(row0 + off, sz)])
            off += sz

    @pl.when(s == NS - 1)
    def _zero_last():
        off = 0
        while off < RPT_LAST:
            sz = min(CH, RPT_LAST - off)
            pltpu.sync_copy(m_buf.at[0].at[pl.ds(0, sz)],
                            agg_sp.at[pl.ds(row0 + off, sz)])
            off += sz

    plsc.subcore_barrier()

    def _valid(u):
        return (s + NS * u) < CHUNKS

    def _base(u):
        return c * E + (s + NS * u) * CH

    def _issue_idx(u, q, b):
        @pl.when(_valid(u))
        def _():
            pltpu.async_copy(src.at[pl.ds(_base(u), CH)], src_v.at[q],
                             sem_i[b])
            pltpu.async_copy(dst.at[pl.ds(_base(u), CH)], dst_v.at[q],
                             sem_i[b])

    def _wait_idx(q, b):
        pltpu.make_async_copy(src.at[pl.ds(0, CH)], src_v.at[q],
                              sem_i[b]).wait()
        pltpu.make_async_copy(dst.at[pl.ds(0, CH)], dst_v.at[q],
                              sem_i[b]).wait()

    def _issue_ew(u, b):
        @pl.when(_valid(u))
        def _():
            pltpu.async_copy(ew.at[pl.ds(_base(u), CH)], ew_buf.at[b],
                             sem_e[b])

    def _issue_gather(q, b):
        pltpu.async_copy(xw.at[src_v.at[q]], g_buf.at[b], sem_g)

    def _outer_body(tt):
        for bi in range(4):
            t = 4 * tt + bi
            b = bi % 2
            nb = 1 - b
            q1 = (bi + 1) % 4
            q2 = (bi + 2) % 4

            @pl.when(_valid(t))
            def _wait_cur():
                pltpu.make_async_copy(ew.at[pl.ds(0, CH)], ew_buf.at[b],
                                      sem_e[b]).wait()
                pltpu.make_async_copy(xw.at[src_v.at[b]], g_buf.at[b],
                                      sem_g).wait()

            @pl.when(_valid(t + 1))
            def _gather_next():
                _wait_idx(q1, nb)
                _issue_gather(q1, nb)

            @pl.when(jnp.logical_and(t >= 2, _valid(t - 2)))
            def _drain_scatter():
                pltpu.make_async_copy(m_buf.at[b], agg_sp.at[dst_v.at[b]],
                                      sem_s[b]).wait()

            _issue_idx(t + 2, q2, b)

            @pl.when(_valid(t))
            def _relu():
                def _row(i, c2):
                    c16 = jnp.full((16,), 16, jnp.int32)
                    cmask = jnp.full((16,), -65536, jnp.int32)
                    for v in range(H // 32):
                        w32 = ew_buf[b, i, pl.ds(v * 16, 16)]
                        lo = lax.bitcast_convert_type(
                            lax.shift_left(w32, c16), _F32)
                        hi = lax.bitcast_convert_type(
                            lax.bitwise_and(w32, cmask), _F32)
                        sl0 = pl.ds(2 * v * 16, 16)
                        sl1 = pl.ds((2 * v + 1) * 16, 16)
                        m_buf[b, i, sl0] = jnp.maximum(
                            g_buf[b, i, sl0] + lo, 0.0)
                        m_buf[b, i, sl1] = jnp.maximum(
                            g_buf[b, i, sl1] + hi, 0.0)
                    return c2
                lax.fori_loop(0, CH, _row, 0)

            _issue_ew(t + 2, b)

            @pl.when(_valid(t))
            def _scatter():
                pltpu.async_copy(m_buf.at[b], agg_sp.at[dst_v.at[bi]],
                                 sem_s[b], add=True)

    _issue_idx(0, 0, 0)

    @pl.when(_valid(0))
    def _prime0():
        _wait_idx(0, 0)
        _issue_gather(0, 0)
    _issue_ew(0, 0)
    _issue_idx(1, 1, 1)
    _issue_ew(1, 1)

    pl.loop(0, (TRIPS + 4) // 4)(_outer_body)

    plsc.subcore_barrier()

    @pl.when(s < NS - 1)
    def _out_mid():
        pltpu.sync_copy(agg_sp.at[pl.ds(s * RPT, RPT)],
                        agg_out.at[pl.ds(c * N + s * RPT, RPT)])

    @pl.when(s == NS - 1)
    def _out_last():
        pltpu.sync_copy(agg_sp.at[pl.ds(s * RPT, RPT_LAST)],
                        agg_out.at[pl.ds(c * N + s * RPT, RPT_LAST)])


@functools.cache
def _sc_edge_kernel():
    return pl.kernel(
        _sc_edge_body,
        out_type=jax.ShapeDtypeStruct((2 * N, H), _F32),
        mesh=plsc.VectorSubcoreMesh(core_axis_name="c", subcore_axis_name="s",
                                    num_cores=2, num_subcores=NS),
        scratch_types=[
            pltpu.VMEM((4, CH), jnp.int32),
            pltpu.VMEM((4, CH), jnp.int32),
            pltpu.VMEM((2, CH, H // 2), jnp.int32),
            pltpu.VMEM((2, CH, H), _F32),
            pltpu.VMEM((2, CH, H), _F32),
            pltpu.VMEM_SHARED((N, H), _F32),
            pltpu.SemaphoreType.DMA,
            pltpu.SemaphoreType.DMA,
            pltpu.SemaphoreType.DMA,
            pltpu.SemaphoreType.DMA,
            pltpu.SemaphoreType.DMA,
            pltpu.SemaphoreType.DMA,
            pltpu.SemaphoreType.DMA,
        ],
    )


def _sc_edge(xw, ew, src, dst):
    return _sc_edge_kernel()(xw, ew, src, dst)


_NBLK = 2000
_EBLK = 2560


def _dot(a, b):
    return jnp.dot(a, b, preferred_element_type=_F32,
                   precision=lax.Precision.HIGHEST)


def _prep_body(nf, Wn, bn, Wm0t, x0, xw0):
    x = _dot(nf[...], Wn[...]) + bn[...]
    x0[...] = x
    xw0[...] = _dot(x, Wm0t[...])


def _prep(nf, Wn, bn, Wm0t):
    return pl.pallas_call(
        _prep_body,
        grid=(2 * N // _NBLK,),
        in_specs=[
            pl.BlockSpec((_NBLK, H), lambda i: (i, 0)),
            pl.BlockSpec((H, H), lambda i: (0, 0)),
            pl.BlockSpec((1, H), lambda i: (0, 0)),
            pl.BlockSpec((H, H), lambda i: (0, 0)),
        ],
        out_specs=[
            pl.BlockSpec((_NBLK, H), lambda i: (i, 0)),
            pl.BlockSpec((_NBLK, H), lambda i: (i, 0)),
        ],
        out_shape=[jax.ShapeDtypeStruct((2 * N, H), _F32)] * 2,
    )(nf, Wn, bn, Wm0t)


def _bf16_bits(x):
    xi = lax.bitcast_convert_type(x, jnp.int32)
    return lax.bitwise_and(xi + jnp.int32(0x8000), jnp.int32(-65536))


def _ew_body(ef, We, be, WmbL, bmL, WmbH, bmH, ew0, ew1, ew2):
    for l, o in enumerate((ew0, ew1, ew2)):
        WfL = _dot(We[...], WmbL[l])
        bfL = _dot(be[...], WmbL[l]) + bmL[l]
        WfH = _dot(We[...], WmbH[l])
        bfH = _dot(be[...], WmbH[l]) + bmH[l]
        u = _bf16_bits(_dot(ef[...], WfL) + bfL)
        v = _bf16_bits(_dot(ef[...], WfH) + bfH)
        o[...] = lax.bitwise_or(lax.shift_right_logical(u, 16), v)


def _ew_all(ef, We, be, WmbL, bmL, WmbH, bmH):
    wspec = pl.BlockSpec((NL, H, H // 2), lambda i: (0, 0, 0))
    bspec = pl.BlockSpec((NL, 1, H // 2), lambda i: (0, 0, 0))
    return pl.pallas_call(
        _ew_body,
        grid=(2 * E // _EBLK,),
        in_specs=[
            pl.BlockSpec((_EBLK, DB), lambda i: (i, 0)),
            pl.BlockSpec((DB, H), lambda i: (0, 0)),
            pl.BlockSpec((1, H), lambda i: (0, 0)),
            wspec, bspec, wspec, bspec,
        ],
        out_specs=[pl.BlockSpec((_EBLK, H // 2), lambda i: (i, 0))] * 3,
        out_shape=[jax.ShapeDtypeStruct((2 * E, H // 2), jnp.int32)] * 3,
    )(ef, We, be, WmbL, bmL, WmbH, bmH)


def _ln_update(x, agg, Wut, Wub, bu, lg, lb):
    t = x + _dot(x, Wut) + _dot(agg, Wub) + bu
    mu = jnp.mean(t, axis=1, keepdims=True)
    var = jnp.mean((t - mu) ** 2, axis=1, keepdims=True)
    return (t - mu) / jnp.sqrt(var + 1e-5) * lg + lb


def _upd_body(x, agg, Wut, Wub, bu, lg, lb, Wmtn, xo, xwo):
    xn = _ln_update(x[...], agg[...], Wut[...], Wub[...], bu[...], lg[...],
                    lb[...])
    xo[...] = xn
    xwo[...] = _dot(xn, Wmtn[...])


def _upd(x, agg, Wut, Wub, bu, lg, lb, Wmtn):
    wspec = pl.BlockSpec((H, H), lambda i: (0, 0))
    vspec = pl.BlockSpec((1, H), lambda i: (0, 0))
    nspec = pl.BlockSpec((_NBLK, H), lambda i: (i, 0))
    return pl.pallas_call(
        _upd_body,
        grid=(2 * N // _NBLK,),
        in_specs=[nspec, nspec, wspec, wspec, vspec, vspec, vspec, wspec],
        out_specs=[nspec, nspec],
        out_shape=[jax.ShapeDtypeStruct((2 * N, H), _F32)] * 2,
    )(x, agg, Wut, Wub, bu, lg, lb, Wmtn)


def _upd_pool_body(x, agg, Wut, Wub, bu, lg, lb, batch, pooled, counts):
    i = pl.program_id(0)
    xn = _ln_update(x[...], agg[...], Wut[...], Wub[...], bu[...], lg[...],
                    lb[...])
    gid = lax.broadcasted_iota(jnp.int32, (_NBLK, 2 * G), 1)
    oh = (batch[...] == gid).astype(_F32)
    p = lax.dot_general(oh, xn, (((0,), (0,)), ((), ())),
                        preferred_element_type=_F32,
                        precision=lax.Precision.HIGHEST)
    cnt = lax.dot_general(oh, jnp.ones((_NBLK, 1), _F32),
                          (((0,), (0,)), ((), ())),
                          preferred_element_type=_F32,
                          precision=lax.Precision.HIGHEST)

    @pl.when(i == 0)
    def _():
        pooled[...] = jnp.zeros_like(pooled)
        counts[...] = jnp.zeros_like(counts)

    pooled[...] += p
    counts[...] += cnt


def _upd_pool(x, agg, Wut, Wub, bu, lg, lb, batch):
    wspec = pl.BlockSpec((H, H), lambda i: (0, 0))
    vspec = pl.BlockSpec((1, H), lambda i: (0, 0))
    nspec = pl.BlockSpec((_NBLK, H), lambda i: (i, 0))
    return pl.pallas_call(
        _upd_pool_body,
        grid=(2 * N // _NBLK,),
        in_specs=[nspec, nspec, wspec, wspec, vspec, vspec, vspec,
                  pl.BlockSpec((_NBLK, 1), lambda i: (i, 0))],
        out_specs=[pl.BlockSpec((2 * G, H), lambda i: (0, 0)),
                   pl.BlockSpec((2 * G, 1), lambda i: (0, 0))],
        out_shape=[jax.ShapeDtypeStruct((2 * G, H), _F32),
                   jax.ShapeDtypeStruct((2 * G, 1), _F32)],
    )(x, agg, Wut, Wub, bu, lg, lb, batch)


def _head_body(pooled, counts, pW1, pb1, pW2, pb2, Lnd, Ld, hW1dm, hW1d, hW1s,
               hb1, hW2, hb2, hW3, hb3, out):
    mean = pooled[...] / jnp.maximum(counts[...], 1.0)
    h = _dot(jnp.maximum(_dot(mean, pW1[...]) + pb1[...], 0.0), pW2[...]) \
        + pb2[...]
    ha = h[:G]
    hb = h[G:]
    delta = hb - ha
    ssum = ha + hb
    x = Ld[...]
    sp = jnp.maximum(x, 0.0) + jnp.log1p(jnp.exp(-jnp.abs(x))) + 0.01
    r = lax.broadcasted_iota(jnp.int32, (H, H), 0)
    cc = lax.broadcasted_iota(jnp.int32, (H, H), 1)
    Lm = Lnd[...] + jnp.where(r == cc, jnp.broadcast_to(sp, (H, H)),
                              jnp.zeros((H, H), _F32))
    dL = _dot(delta, Lm)
    d_m = jnp.sqrt(jnp.sum(dL * dL, axis=1, keepdims=True) + 1e-8)
    z = jnp.maximum(d_m * hW1dm[...] + _dot(delta, hW1d[...])
                    + _dot(ssum, hW1s[...]) + hb1[...], 0.0)
    z = jnp.maximum(_dot(z, hW2[...]) + hb2[...], 0.0)
    out[...] = _dot(z, hW3[...]) + hb3[...]


def _head(pooled, counts, p, Lnd):
    args = (pooled, counts, p['pW1'], p['pb1'].reshape(1, H), p['pW2'],
            p['pb2'].reshape(1, H), Lnd, p['L_diag'].reshape(1, H),
            p['hW1'][0:1], p['hW1'][1:H + 1], p['hW1'][H + 1:],
            p['hb1'].reshape(1, H), p['hW2'], p['hb2'].reshape(1, H // 2),
            p['hW3'], p['hb3'].reshape(1, 1))
    return pl.pallas_call(
        _head_body,
        out_shape=jax.ShapeDtypeStruct((G, 1), _F32),
    )(*args)


_TRIL_R, _TRIL_C = np.tril_indices(H, -1)

_K = np.arange(H // 2)
_PERM_LO = 32 * (_K // 16) + _K % 16
_PERM_HI = _PERM_LO + 16


def kernel(node_feats_a, edge_feats_a, edge_index_a, batch_a, node_feats_b,
           edge_feats_b, edge_index_b, batch_b, params):
    p = params
    nf = jnp.concatenate([node_feats_a, node_feats_b], axis=0)
    ef = jnp.concatenate([edge_feats_a, edge_feats_b], axis=0)
    src = jnp.concatenate([edge_index_a[0], edge_index_b[0] + N], axis=0)
    dst = jnp.concatenate([edge_index_a[1], edge_index_b[1]], axis=0)
    batch = jnp.concatenate([batch_a, batch_b + G], axis=0).reshape(2 * N, 1)

    WmbL = jnp.stack([p['Wm%d' % l][H:][:, _PERM_LO] for l in range(NL)])
    bmL = jnp.stack([p['bm%d' % l][_PERM_LO].reshape(1, H // 2)
                     for l in range(NL)])
    WmbH = jnp.stack([p['Wm%d' % l][H:][:, _PERM_HI] for l in range(NL)])
    bmH = jnp.stack([p['bm%d' % l][_PERM_HI].reshape(1, H // 2)
                     for l in range(NL)])
    Lnd = jnp.zeros((H, H), _F32).at[_TRIL_R, _TRIL_C].set(p['L_lower'])

    x, xw = _prep(nf, p['Wn'], p['bn'].reshape(1, H), p['Wm0'][:H])
    ews = _ew_all(ef, p['We'], p['be'].reshape(1, H), WmbL, bmL, WmbH, bmH)

    for l in range(NL):
        agg = _sc_edge(xw, ews[l], src, dst)
        if l + 1 < NL:
            x, xw = _upd(x, agg, p['Wu%d' % l][:H], p['Wu%d' % l][H:],
                         p['bu%d' % l].reshape(1, H),
                         p['lg%d' % l].reshape(1, H),
                         p['lb%d' % l].reshape(1, H),
                         p['Wm%d' % (l + 1)][:H])
        else:
            pooled, counts = _upd_pool(x, agg, p['Wu%d' % l][:H],
                                       p['Wu%d' % l][H:],
                                       p['bu%d' % l].reshape(1, H),
                                       p['lg%d' % l].reshape(1, H),
                                       p['lb%d' % l].reshape(1, H), batch)

    out = _head(pooled, counts, p, Lnd)
    return out[:, 0]


# speedup vs baseline: 1.7247x; 1.7247x over previous
import functools

import numpy as np
import jax
import jax.numpy as jnp
from jax import lax
from jax.experimental import pallas as pl
from jax.experimental.pallas import tpu as pltpu
from jax.experimental.pallas import tpu_sc as plsc

N = 10000
E = 320000
H = 128
DB = 16
G = 256
NL = 3

NS = 16
CH = 64
CHUNKS = E // CH
TRIPS = (CHUNKS + NS - 1) // NS
RPT = 624
RPT_LAST = N - (NS - 1) * RPT

_F32 = jnp.float32


def _sc_edge_body(xw, ew, src, dst, agg_out, src_v, dst_v, ew_buf, g_buf,
                  m_buf, agg_sp, sem_i0, sem_i1, sem_e0, sem_e1, sem_g,
                  sem_s0, sem_s1):
    c = lax.axis_index("c")
    s = lax.axis_index("s")
    sem_i = (sem_i0, sem_i1)
    sem_e = (sem_e0, sem_e1)
    sem_s = (sem_s0, sem_s1)

    def _zrow(i, carry):
        for v in range(H // 16):
            m_buf[0, i, pl.ds(v * 16, 16)] = jnp.zeros((16,), _F32)
        return carry
    lax.fori_loop(0, CH, _zrow, 0)
    row0 = s * RPT

    @pl.when(s < NS - 1)
    def _zero_mid():
        off = 0
        while off < RPT:
            sz = min(CH, RPT - off)
            pltpu.sync_copy(m_buf.at[0].at[pl.ds(0, sz)],
                            agg_sp.at[pl.ds(row0 + off, sz)])
            off += sz

    @pl.when(s == NS - 1)
    def _zero_last():
        off = 0
        while off < RPT_LAST:
            sz = min(CH, RPT_LAST - off)
            pltpu.sync_copy(m_buf.at[0].at[pl.ds(0, sz)],
                            agg_sp.at[pl.ds(row0 + off, sz)])
            off += sz

    plsc.subcore_barrier()

    def _valid(u):
        return (s + NS * u) < CHUNKS

    def _base(u):
        return c * E + (s + NS * u) * CH

    def _issue_idx(u, q, b):
        @pl.when(_valid(u))
        def _():
            pltpu.async_copy(src.at[pl.ds(_base(u), CH)], src_v.at[q],
                             sem_i[b])
            pltpu.async_copy(dst.at[pl.ds(_base(u), CH)], dst_v.at[q],
                             sem_i[b])

    def _wait_idx(q, b):
        pltpu.make_async_copy(src.at[pl.ds(0, CH)], src_v.at[q],
                              sem_i[b]).wait()
        pltpu.make_async_copy(dst.at[pl.ds(0, CH)], dst_v.at[q],
                              sem_i[b]).wait()

    def _issue_ew(u, b):
        @pl.when(_valid(u))
        def _():
            pltpu.async_copy(ew.at[pl.ds(_base(u), CH)], ew_buf.at[b],
                             sem_e[b])

    def _issue_gather(q, b):
        pltpu.async_copy(xw.at[src_v.at[q]], g_buf.at[b], sem_g)

    def _outer_body(tt):
        for bi in range(4):
            t = 4 * tt + bi
            b = bi % 2
            nb = 1 - b
            q1 = (bi + 1) % 4
            q2 = (bi + 2) % 4

            @pl.when(_valid(t))
            def _wait_cur():
                pltpu.make_async_copy(ew.at[pl.ds(0, CH)], ew_buf.at[b],
                                      sem_e[b]).wait()
                pltpu.make_async_copy(xw.at[src_v.at[b]], g_buf.at[b],
                                      sem_g).wait()

            @pl.when(_valid(t + 1))
            def _gather_next():
                _wait_idx(q1, nb)
                _issue_gather(q1, nb)

            @pl.when(jnp.logical_and(t >= 2, _valid(t - 2)))
            def _drain_scatter():
                pltpu.make_async_copy(m_buf.at[b], agg_sp.at[dst_v.at[b]],
                                      sem_s[b]).wait()

            _issue_idx(t + 2, q2, b)

            @pl.when(_valid(t))
            def _relu():
                def _row(i, c2):
                    for v in range(H // 16):
                        sl = pl.ds(v * 16, 16)
                        m_buf[b, i, sl] = jnp.maximum(
                            g_buf[b, i, sl] + ew_buf[b, i, sl], 0.0)
                    return c2
                lax.fori_loop(0, CH, _row, 0)

            _issue_ew(t + 2, b)

            @pl.when(_valid(t))
            def _scatter():
                pltpu.async_copy(m_buf.at[b], agg_sp.at[dst_v.at[bi]],
                                 sem_s[b], add=True)

    _issue_idx(0, 0, 0)

    @pl.when(_valid(0))
    def _prime0():
        _wait_idx(0, 0)
        _issue_gather(0, 0)
    _issue_ew(0, 0)
    _issue_idx(1, 1, 1)
    _issue_ew(1, 1)

    pl.loop(0, (TRIPS + 4) // 4)(_outer_body)

    plsc.subcore_barrier()

    @pl.when(s < NS - 1)
    def _out_mid():
        pltpu.sync_copy(agg_sp.at[pl.ds(s * RPT, RPT)],
                        agg_out.at[pl.ds(c * N + s * RPT, RPT)])

    @pl.when(s == NS - 1)
    def _out_last():
        pltpu.sync_copy(agg_sp.at[pl.ds(s * RPT, RPT_LAST)],
                        agg_out.at[pl.ds(c * N + s * RPT, RPT_LAST)])


@functools.cache
def _sc_edge_kernel():
    return pl.kernel(
        _sc_edge_body,
        out_type=jax.ShapeDtypeStruct((2 * N, H), _F32),
        mesh=plsc.VectorSubcoreMesh(core_axis_name="c", subcore_axis_name="s",
                                    num_cores=2, num_subcores=NS),
        scratch_types=[
            pltpu.VMEM((4, CH), jnp.int32),
            pltpu.VMEM((4, CH), jnp.int32),
            pltpu.VMEM((2, CH, H), _F32),
            pltpu.VMEM((2, CH, H), _F32),
            pltpu.VMEM((2, CH, H), _F32),
            pltpu.VMEM_SHARED((N, H), _F32),
            pltpu.SemaphoreType.DMA,
            pltpu.SemaphoreType.DMA,
            pltpu.SemaphoreType.DMA,
            pltpu.SemaphoreType.DMA,
            pltpu.SemaphoreType.DMA,
            pltpu.SemaphoreType.DMA,
            pltpu.SemaphoreType.DMA,
        ],
    )


def _sc_edge(xw, ew, src, dst):
    return _sc_edge_kernel()(xw, ew, src, dst)


_NBLK = 2000
_EBLK = 2560


def _dot(a, b):
    return jnp.dot(a, b, preferred_element_type=_F32,
                   precision=lax.Precision.HIGHEST)


def _dot_hi(a, b):
    return jnp.dot(a, b, preferred_element_type=_F32,
                   precision=lax.Precision.DEFAULT)


_NHALF = N // _NBLK


def _prep_body(nfa, nfb, Wn, bn, Wm0t, x0, xw0):
    i = pl.program_id(0)

    def _emit(nf):
        x = _dot(nf, Wn[...]) + bn[...]
        x0[...] = x
        xw0[...] = _dot(x, Wm0t[...])

    @pl.when(i < _NHALF)
    def _():
        _emit(nfa[...])

    @pl.when(i >= _NHALF)
    def _():
        _emit(nfb[...])


def _prep(nfa, nfb, Wn, bn, Wm0t):
    return pl.pallas_call(
        _prep_body,
        grid=(2 * N // _NBLK,),
        in_specs=[
            pl.BlockSpec((_NBLK, H), lambda i: (jnp.minimum(i, _NHALF - 1), 0)),
            pl.BlockSpec((_NBLK, H),
                         lambda i: (jnp.maximum(i - _NHALF, 0), 0)),
            pl.BlockSpec((H, H), lambda i: (0, 0)),
            pl.BlockSpec((1, H), lambda i: (0, 0)),
            pl.BlockSpec((H, H), lambda i: (0, 0)),
        ],
        out_specs=[
            pl.BlockSpec((_NBLK, H), lambda i: (i, 0)),
            pl.BlockSpec((_NBLK, H), lambda i: (i, 0)),
        ],
        out_shape=[jax.ShapeDtypeStruct((2 * N, H), _F32)] * 2,
    )(nfa, nfb, Wn, bn, Wm0t)


_EHALF = E // _EBLK


def _ew_body(efa, efb, We, be, Wmb, bm, ew0, ew1, ew2):
    i = pl.program_id(0)

    def _emit(ef):
        for l, o in enumerate((ew0, ew1, ew2)):
            Wf = _dot(We[...], Wmb[l])
            bf = _dot(be[...], Wmb[l]) + bm[l]
            o[...] = _dot_hi(ef, Wf) + bf

    @pl.when(i < _EHALF)
    def _():
        _emit(efa[...])

    @pl.when(i >= _EHALF)
    def _():
        _emit(efb[...])


def _ew_all(efa, efb, We, be, Wmb, bm):
    return pl.pallas_call(
        _ew_body,
        grid=(2 * E // _EBLK,),
        in_specs=[
            pl.BlockSpec((_EBLK, DB), lambda i: (jnp.minimum(i, _EHALF - 1),
                                                 0)),
            pl.BlockSpec((_EBLK, DB), lambda i: (jnp.maximum(i - _EHALF, 0),
                                                 0)),
            pl.BlockSpec((DB, H), lambda i: (0, 0)),
            pl.BlockSpec((1, H), lambda i: (0, 0)),
            pl.BlockSpec((NL, H, H), lambda i: (0, 0, 0)),
            pl.BlockSpec((NL, 1, H), lambda i: (0, 0, 0)),
        ],
        out_specs=[pl.BlockSpec((_EBLK, H), lambda i: (i, 0))] * 3,
        out_shape=[jax.ShapeDtypeStruct((2 * E, H), _F32)] * 3,
    )(efa, efb, We, be, Wmb, bm)


def _ln_update(x, agg, Wut, Wub, bu, lg, lb):
    t = x + _dot(x, Wut) + _dot(agg, Wub) + bu
    mu = jnp.mean(t, axis=1, keepdims=True)
    var = jnp.mean((t - mu) ** 2, axis=1, keepdims=True)
    return (t - mu) / jnp.sqrt(var + 1e-5) * lg + lb


def _upd_body(x, agg, Wut, Wub, bu, lg, lb, Wmtn, xo, xwo):
    xn = _ln_update(x[...], agg[...], Wut[...], Wub[...], bu[...], lg[...],
                    lb[...])
    xo[...] = xn
    xwo[...] = _dot(xn, Wmtn[...])


def _upd(x, agg, Wut, Wub, bu, lg, lb, Wmtn):
    wspec = pl.BlockSpec((H, H), lambda i: (0, 0))
    vspec = pl.BlockSpec((1, H), lambda i: (0, 0))
    nspec = pl.BlockSpec((_NBLK, H), lambda i: (i, 0))
    return pl.pallas_call(
        _upd_body,
        grid=(2 * N // _NBLK,),
        in_specs=[nspec, nspec, wspec, wspec, vspec, vspec, vspec, wspec],
        out_specs=[nspec, nspec],
        out_shape=[jax.ShapeDtypeStruct((2 * N, H), _F32)] * 2,
    )(x, agg, Wut, Wub, bu, lg, lb, Wmtn)


def _upd_pool_body(x, agg, Wut, Wub, bu, lg, lb, batch, pooled, counts):
    i = pl.program_id(0)
    xn = _ln_update(x[...], agg[...], Wut[...], Wub[...], bu[...], lg[...],
                    lb[...])
    gid = lax.broadcasted_iota(jnp.int32, (_NBLK, 2 * G), 1)
    oh = (batch[...] == gid).astype(_F32)
    p = lax.dot_general(oh, xn, (((0,), (0,)), ((), ())),
                        preferred_element_type=_F32,
                        precision=lax.Precision.DEFAULT)
    cnt = lax.dot_general(oh, jnp.ones((_NBLK, 1), _F32),
                          (((0,), (0,)), ((), ())),
                          preferred_element_type=_F32,
                          precision=lax.Precision.DEFAULT)

    @pl.when(i == 0)
    def _():
        pooled[...] = jnp.zeros_like(pooled)
        counts[...] = jnp.zeros_like(counts)

    pooled[...] += p
    counts[...] += cnt


def _upd_pool(x, agg, Wut, Wub, bu, lg, lb, batch):
    wspec = pl.BlockSpec((H, H), lambda i: (0, 0))
    vspec = pl.BlockSpec((1, H), lambda i: (0, 0))
    nspec = pl.BlockSpec((_NBLK, H), lambda i: (i, 0))
    return pl.pallas_call(
        _upd_pool_body,
        grid=(2 * N // _NBLK,),
        in_specs=[nspec, nspec, wspec, wspec, vspec, vspec, vspec,
                  pl.BlockSpec((_NBLK, 1), lambda i: (i, 0))],
        out_specs=[pl.BlockSpec((2 * G, H), lambda i: (0, 0)),
                   pl.BlockSpec((2 * G, 1), lambda i: (0, 0))],
        out_shape=[jax.ShapeDtypeStruct((2 * G, H), _F32),
                   jax.ShapeDtypeStruct((2 * G, 1), _F32)],
    )(x, agg, Wut, Wub, bu, lg, lb, batch)


def _head_body(pooled, counts, pW1, pb1, pW2, pb2, Lnd, Ld, hW1dm, hW1d, hW1s,
               hb1, hW2, hb2, hW3, hb3, out):
    mean = pooled[...] / jnp.maximum(counts[...], 1.0)
    h = _dot(jnp.maximum(_dot(mean, pW1[...]) + pb1[...], 0.0), pW2[...]) \
        + pb2[...]
    ha = h[:G]
    hb = h[G:]
    delta = hb - ha
    ssum = ha + hb
    x = Ld[...]
    sp = jnp.maximum(x, 0.0) + jnp.log1p(jnp.exp(-jnp.abs(x))) + 0.01
    r = lax.broadcasted_iota(jnp.int32, (H, H), 0)
    cc = lax.broadcasted_iota(jnp.int32, (H, H), 1)
    Lm = Lnd[...] + jnp.where(r == cc, jnp.broadcast_to(sp, (H, H)),
                              jnp.zeros((H, H), _F32))
    dL = _dot(delta, Lm)
    d_m = jnp.sqrt(jnp.sum(dL * dL, axis=1, keepdims=True) + 1e-8)
    z = jnp.maximum(d_m * hW1dm[...] + _dot(delta, hW1d[...])
                    + _dot(ssum, hW1s[...]) + hb1[...], 0.0)
    z = jnp.maximum(_dot(z, hW2[...]) + hb2[...], 0.0)
    out[...] = _dot(z, hW3[...]) + hb3[...]


def _head(pooled, counts, p, Lnd):
    args = (pooled, counts, p['pW1'], p['pb1'].reshape(1, H), p['pW2'],
            p['pb2'].reshape(1, H), Lnd, p['L_diag'].reshape(1, H),
            p['hW1'][0:1], p['hW1'][1:H + 1], p['hW1'][H + 1:],
            p['hb1'].reshape(1, H), p['hW2'], p['hb2'].reshape(1, H // 2),
            p['hW3'], p['hb3'].reshape(1, 1))
    return pl.pallas_call(
        _head_body,
        out_shape=jax.ShapeDtypeStruct((G, 1), _F32),
    )(*args)


_TRIL_R, _TRIL_C = np.tril_indices(H, -1)


def kernel(node_feats_a, edge_feats_a, edge_index_a, batch_a, node_feats_b,
           edge_feats_b, edge_index_b, batch_b, params):
    p = params
    src = jnp.concatenate([edge_index_a[0], edge_index_b[0] + N], axis=0)
    dst = jnp.concatenate([edge_index_a[1], edge_index_b[1]], axis=0)
    batch = jnp.concatenate([batch_a, batch_b + G], axis=0).reshape(2 * N, 1)

    Wmb = jnp.stack([p['Wm%d' % l][H:] for l in range(NL)])
    bm = jnp.stack([p['bm%d' % l].reshape(1, H) for l in range(NL)])
    Lnd = jnp.zeros((H, H), _F32).at[_TRIL_R, _TRIL_C].set(p['L_lower'])

    x, xw = _prep(node_feats_a, node_feats_b, p['Wn'], p['bn'].reshape(1, H),
                  p['Wm0'][:H])
    ews = _ew_all(edge_feats_a, edge_feats_b, p['We'], p['be'].reshape(1, H),
                  Wmb, bm)

    for l in range(NL):
        agg = _sc_edge(xw, ews[l], src, dst)
        if l + 1 < NL:
            x, xw = _upd(x, agg, p['Wu%d' % l][:H], p['Wu%d' % l][H:],
                         p['bu%d' % l].reshape(1, H),
                         p['lg%d' % l].reshape(1, H),
                         p['lb%d' % l].reshape(1, H),
                         p['Wm%d' % (l + 1)][:H])
        else:
            pooled, counts = _upd_pool(x, agg, p['Wu%d' % l][:H],
                                       p['Wu%d' % l][H:],
                                       p['bu%d' % l].reshape(1, H),
                                       p['lg%d' % l].reshape(1, H),
                                       p['lb%d' % l].reshape(1, H), batch)

    out = _head(pooled, counts, p, Lnd)
    return out[:, 0]
